# baseline (speedup 1.0000x reference)
import functools

import jax
import jax.numpy as jnp
from jax import lax
from jax.experimental import pallas as pl
from jax.experimental.pallas import tpu as pltpu
from jax.experimental.pallas import tpu_sc as plsc


_NB = 64
_LANES = 128
_SUB = 8


def _tile_body(t_ref, o_ref):
    x = t_ref[...]
    nb = x.shape[1] // _LANES
    for a in range(_SUB):
        xa = x[_SUB * a:_SUB * (a + 1), :]
        za = jnp.swapaxes(xa.reshape(_SUB, nb, _LANES), 0, 1)
        o_ref[a, :, :, :] = za + za


def _to_tiles(t_t):
    d, m = t_t.shape
    tb = pl.cdiv(m, _LANES)
    grid = pl.cdiv(tb, _NB)
    return pl.pallas_call(
        _tile_body,
        grid=(grid,),
        in_specs=[pl.BlockSpec((d, _LANES * _NB), lambda i: (0, i))],
        out_specs=pl.BlockSpec((_SUB, _NB, _SUB, _LANES),
                               lambda i: (0, i, 0, 0)),
        out_shape=jax.ShapeDtypeStruct((_SUB, tb, _SUB, _LANES), jnp.float32),
        compiler_params=pltpu.CompilerParams(
            dimension_semantics=("arbitrary",)),
    )(t_t)


def _untile_body(z_ref, o_ref):
    nb = z_ref.shape[1]
    for a in range(_SUB):
        za = z_ref[a, :, :, :]
        o_ref[_SUB * a:_SUB * (a + 1), :] = (
            jnp.swapaxes(za, 0, 1).reshape(_SUB, nb * _LANES))


def _from_tiles(z, m):
    s, tb, s2, lanes = z.shape
    d = s * s2
    grid = pl.cdiv(tb, _NB)
    return pl.pallas_call(
        _untile_body,
        grid=(grid,),
        in_specs=[pl.BlockSpec((_SUB, _NB, _SUB, _LANES),
                               lambda i: (0, i, 0, 0))],
        out_specs=pl.BlockSpec((d, _LANES * _NB), lambda i: (0, i)),
        out_shape=jax.ShapeDtypeStruct((d, m), jnp.float32),
        compiler_params=pltpu.CompilerParams(
            dimension_semantics=("arbitrary",)),
    )(z)


_JB = 512


def _delta_body(idx_col_ref, idx_row_ref, valt_ref, o_ref):
    eq = idx_col_ref[...] == idx_row_ref[...]
    e = jnp.where(eq, jnp.float32(2.0), jnp.float32(0.0))
    o_ref[...] = lax.dot_general(
        valt_ref[...], e, (((1,), (0,)), ((), ())),
        precision=lax.Precision.HIGHEST,
        preferred_element_type=jnp.float32,
    )


def _delta_t(indices, val_t):
    d, b = val_t.shape
    jb = _JB if b % _JB == 0 else b
    grid = b // jb
    idx_col = indices.reshape(b, 1)
    idx_row = indices.reshape(1, b)
    return pl.pallas_call(
        _delta_body,
        grid=(grid,),
        in_specs=[
            pl.BlockSpec((b, 1), lambda i: (0, 0)),
            pl.BlockSpec((1, jb), lambda i: (0, i)),
            pl.BlockSpec((d, b), lambda i: (0, 0)),
        ],
        out_specs=pl.BlockSpec((d, jb), lambda i: (0, i)),
        out_shape=jax.ShapeDtypeStruct((d, b), jnp.float32),
        compiler_params=pltpu.CompilerParams(
            dimension_semantics=("arbitrary",)),
    )(idx_col, idx_row, val_t)


_IDXW = 128


def _make_sc_body(a_stride):
    def _sc_body(zref, deltat_hbm, idx_hbm, idxv, base_a, base_b,
                 addr_a, addr_b, words_a, words_b, dv_a, dv_b, sem):
        c = lax.axis_index("c")
        s = lax.axis_index("s")

        @pl.when(c == 0)
        def _():
            pltpu.sync_copy(idx_hbm.at[pl.ds(2 * s, 2)], idxv)
            for ci, (base, dv) in enumerate(((base_a, dv_a),
                                             (base_b, dv_b))):
                for j in range(_IDXW // 16):
                    sl = pl.ds(16 * j, 16)
                    v = idxv[ci, sl]
                    base[sl] = ((lax.shift_right_logical(v, 7) * 1024)
                                + (v & 127))
                pltpu.sync_copy(
                    deltat_hbm.at[:, pl.ds((2 * s + ci) * _IDXW, _IDXW)], dv)

            def fill(aq, _):
                off = (aq // 8) * a_stride + (aq % 8) * 128
                for j in range(_IDXW // 16):
                    sl = pl.ds(16 * j, 16)
                    addr_a[aq, sl] = base_a[sl] + off
                    addr_b[aq, sl] = base_b[sl] + off
                return 0

            lax.fori_loop(0, 64, fill, 0)

            def gat(aq, _):
                pltpu.async_copy(zref.at[addr_a.at[aq]], words_a.at[aq], sem)
                pltpu.async_copy(zref.at[addr_b.at[aq]], words_b.at[aq], sem)
                return 0

            lax.fori_loop(0, 64, gat, 0)
            pltpu.make_async_copy(deltat_hbm.at[:, 0:_IDXW], words_a, sem).wait()
            pltpu.make_async_copy(deltat_hbm.at[:, 0:_IDXW], words_b, sem).wait()

            def add(aq, _):
                for j in range(_IDXW // 16):
                    sl = pl.ds(16 * j, 16)
                    words_a[aq, sl] = words_a[aq, sl] + dv_a[aq, sl]
                    words_b[aq, sl] = words_b[aq, sl] + dv_b[aq, sl]
                return 0

            lax.fori_loop(0, 64, add, 0)

            plsc.subcore_barrier()

            def sca(aq, _):
                pltpu.async_copy(words_a.at[aq], zref.at[addr_a.at[aq]], sem)
                pltpu.async_copy(words_b.at[aq], zref.at[addr_b.at[aq]], sem)
                return 0

            lax.fori_loop(0, 64, sca, 0)
            pltpu.make_async_copy(deltat_hbm.at[:, 0:_IDXW], words_a, sem).wait()
            pltpu.make_async_copy(deltat_hbm.at[:, 0:_IDXW], words_b, sem).wait()

    return _sc_body


def _sc_scatter(zref, delta_t, indices, tb):
    d, b = delta_t.shape
    idx2d = indices.reshape(b // _IDXW, _IDXW)
    mesh = plsc.VectorSubcoreMesh(
        core_axis_name="c", subcore_axis_name="s", num_cores=2, num_subcores=16)
    run = pl.kernel(
        _make_sc_body(tb * 1024),
        out_type=(),
        mesh=mesh,
        scratch_types=[
            pltpu.VMEM((2, _IDXW), jnp.int32),
            pltpu.VMEM((_IDXW,), jnp.int32),
            pltpu.VMEM((_IDXW,), jnp.int32),
            pltpu.VMEM((64, _IDXW), jnp.int32),
            pltpu.VMEM((64, _IDXW), jnp.int32),
            pltpu.VMEM((64, _IDXW), jnp.float32),
            pltpu.VMEM((64, _IDXW), jnp.float32),
            pltpu.VMEM((64, _IDXW), jnp.float32),
            pltpu.VMEM((64, _IDXW), jnp.float32),
            pltpu.SemaphoreType.DMA,
        ],
        compiler_params=pltpu.CompilerParams(use_tc_tiling_on_sc=False),
    )
    run(zref, delta_t, idx2d)


def kernel(tensor, val, indices):
    m, d = tensor.shape
    tb = (m + _LANES - 1) // _LANES
    z = _to_tiles(tensor.T)
    delta_t = _delta_t(indices, val.T)
    ref = jax.new_ref(z.reshape(-1))
    _sc_scatter(ref, delta_t, indices, tb)
    zs = jax.freeze(ref).reshape(8, tb, 8, _LANES)
    out_t = _from_tiles(zs, m)
    return out_t.T


# speedup vs baseline: 1.0007x; 1.0007x over previous
import functools

import jax
import jax.numpy as jnp
from jax import lax
from jax.experimental import pallas as pl
from jax.experimental.pallas import tpu as pltpu
from jax.experimental.pallas import tpu_sc as plsc


_NB = 64
_LANES = 128
_SUB = 8


def _tile_body(t_ref, o_ref):
    x = t_ref[...]
    nb = x.shape[1] // _LANES
    for a in range(_SUB):
        xa = x[_SUB * a:_SUB * (a + 1), :]
        za = jnp.swapaxes(xa.reshape(_SUB, nb, _LANES), 0, 1)
        o_ref[a, :, :, :] = za + za


def _to_tiles(t_t):
    d, m = t_t.shape
    tb = pl.cdiv(m, _LANES)
    grid = pl.cdiv(tb, _NB)
    return pl.pallas_call(
        _tile_body,
        grid=(grid,),
        in_specs=[pl.BlockSpec((d, _LANES * _NB), lambda i: (0, i))],
        out_specs=pl.BlockSpec((_SUB, _NB, _SUB, _LANES),
                               lambda i: (0, i, 0, 0)),
        out_shape=jax.ShapeDtypeStruct((_SUB, tb, _SUB, _LANES), jnp.float32),
        compiler_params=pltpu.CompilerParams(
            dimension_semantics=("arbitrary",)),
    )(t_t)


def _untile_body(z_ref, o_ref):
    nb = z_ref.shape[1]
    for a in range(_SUB):
        za = z_ref[a, :, :, :]
        o_ref[_SUB * a:_SUB * (a + 1), :] = (
            jnp.swapaxes(za, 0, 1).reshape(_SUB, nb * _LANES))


def _from_tiles(z, m):
    s, tb, s2, lanes = z.shape
    d = s * s2
    grid = pl.cdiv(tb, _NB)
    return pl.pallas_call(
        _untile_body,
        grid=(grid,),
        in_specs=[pl.BlockSpec((_SUB, _NB, _SUB, _LANES),
                               lambda i: (0, i, 0, 0))],
        out_specs=pl.BlockSpec((d, _LANES * _NB), lambda i: (0, i)),
        out_shape=jax.ShapeDtypeStruct((d, m), jnp.float32),
        compiler_params=pltpu.CompilerParams(
            dimension_semantics=("arbitrary",)),
    )(z)


_JB = 512


def _delta_body(idx_col_ref, idx_row_ref, valt_ref, o_ref):
    eq = idx_col_ref[...] == idx_row_ref[...]
    e = jnp.where(eq, jnp.float32(2.0), jnp.float32(0.0))
    o_ref[...] = lax.dot_general(
        valt_ref[...], e, (((1,), (0,)), ((), ())),
        precision=lax.Precision.HIGHEST,
        preferred_element_type=jnp.float32,
    )


def _delta_t(indices, val_t):
    d, b = val_t.shape
    jb = _JB if b % _JB == 0 else b
    grid = b // jb
    idx_col = indices.reshape(b, 1)
    idx_row = indices.reshape(1, b)
    return pl.pallas_call(
        _delta_body,
        grid=(grid,),
        in_specs=[
            pl.BlockSpec((b, 1), lambda i: (0, 0)),
            pl.BlockSpec((1, jb), lambda i: (0, i)),
            pl.BlockSpec((d, b), lambda i: (0, 0)),
        ],
        out_specs=pl.BlockSpec((d, jb), lambda i: (0, i)),
        out_shape=jax.ShapeDtypeStruct((d, b), jnp.float32),
        compiler_params=pltpu.CompilerParams(
            dimension_semantics=("arbitrary",)),
    )(idx_col, idx_row, val_t)


_IDXW = 128


def _make_sc_body(a_stride):
    def _sc_body(zref, deltat_hbm, idx_hbm, idxv, base_a, base_b,
                 addr_a, addr_b, words_a, words_b, dv_a, dv_b, sem):
        c = lax.axis_index("c")
        s = lax.axis_index("s")

        @pl.when(c == 0)
        def _():
            pltpu.sync_copy(idx_hbm.at[pl.ds(2 * s, 2)], idxv)
            for ci, (base, dv) in enumerate(((base_a, dv_a),
                                             (base_b, dv_b))):
                for j in range(_IDXW // 16):
                    sl = pl.ds(16 * j, 16)
                    v = idxv[ci, sl]
                    base[sl] = ((lax.shift_right_logical(v, 7) * 1024)
                                + (v & 127))
                pltpu.sync_copy(
                    deltat_hbm.at[:, pl.ds((2 * s + ci) * _IDXW, _IDXW)], dv)

            def fill(aq, _):
                off = (aq // 8) * a_stride + (aq % 8) * 128
                for j in range(_IDXW // 16):
                    sl = pl.ds(16 * j, 16)
                    addr_a[aq, sl] = base_a[sl] + off
                    addr_b[aq, sl] = base_b[sl] + off
                return 0

            lax.fori_loop(0, 64, fill, 0)

            @plsc.parallel_loop(0, 64, unroll=8)
            def gat(aq):
                pltpu.async_copy(zref.at[addr_a.at[aq]], words_a.at[aq], sem)
                pltpu.async_copy(zref.at[addr_b.at[aq]], words_b.at[aq], sem)
            pltpu.make_async_copy(deltat_hbm.at[:, 0:_IDXW], words_a, sem).wait()
            pltpu.make_async_copy(deltat_hbm.at[:, 0:_IDXW], words_b, sem).wait()

            def add(aq, _):
                for j in range(_IDXW // 16):
                    sl = pl.ds(16 * j, 16)
                    words_a[aq, sl] = words_a[aq, sl] + dv_a[aq, sl]
                    words_b[aq, sl] = words_b[aq, sl] + dv_b[aq, sl]
                return 0

            lax.fori_loop(0, 64, add, 0)

            plsc.subcore_barrier()

            @plsc.parallel_loop(0, 64, unroll=8)
            def sca(aq):
                pltpu.async_copy(words_a.at[aq], zref.at[addr_a.at[aq]], sem)
                pltpu.async_copy(words_b.at[aq], zref.at[addr_b.at[aq]], sem)
            pltpu.make_async_copy(deltat_hbm.at[:, 0:_IDXW], words_a, sem).wait()
            pltpu.make_async_copy(deltat_hbm.at[:, 0:_IDXW], words_b, sem).wait()

    return _sc_body


def _sc_scatter(zref, delta_t, indices, tb):
    d, b = delta_t.shape
    idx2d = indices.reshape(b // _IDXW, _IDXW)
    mesh = plsc.VectorSubcoreMesh(
        core_axis_name="c", subcore_axis_name="s", num_cores=2, num_subcores=16)
    run = pl.kernel(
        _make_sc_body(tb * 1024),
        out_type=(),
        mesh=mesh,
        scratch_types=[
            pltpu.VMEM((2, _IDXW), jnp.int32),
            pltpu.VMEM((_IDXW,), jnp.int32),
            pltpu.VMEM((_IDXW,), jnp.int32),
            pltpu.VMEM((64, _IDXW), jnp.int32),
            pltpu.VMEM((64, _IDXW), jnp.int32),
            pltpu.VMEM((64, _IDXW), jnp.float32),
            pltpu.VMEM((64, _IDXW), jnp.float32),
            pltpu.VMEM((64, _IDXW), jnp.float32),
            pltpu.VMEM((64, _IDXW), jnp.float32),
            pltpu.SemaphoreType.DMA,
        ],
        compiler_params=pltpu.CompilerParams(use_tc_tiling_on_sc=False),
    )
    run(zref, delta_t, idx2d)


def kernel(tensor, val, indices):
    m, d = tensor.shape
    tb = (m + _LANES - 1) // _LANES
    z = _to_tiles(tensor.T)
    delta_t = _delta_t(indices, val.T)
    ref = jax.new_ref(z.reshape(-1))
    _sc_scatter(ref, delta_t, indices, tb)
    zs = jax.freeze(ref).reshape(8, tb, 8, _LANES)
    out_t = _from_tiles(zs, m)
    return out_t.T


# speedup vs baseline: 1.0079x; 1.0072x over previous
import functools

import jax
import jax.numpy as jnp
from jax import lax
from jax.experimental import pallas as pl
from jax.experimental.pallas import tpu as pltpu
from jax.experimental.pallas import tpu_sc as plsc


_NB = 64
_LANES = 128
_SUB = 8


def _tile_body(t_ref, o_ref):
    x = t_ref[...]
    nb = x.shape[1] // _LANES
    for a in range(_SUB):
        xa = x[_SUB * a:_SUB * (a + 1), :]
        za = jnp.swapaxes(xa.reshape(_SUB, nb, _LANES), 0, 1)
        o_ref[a, :, :, :] = za + za


def _to_tiles(t_t):
    d, m = t_t.shape
    tb = pl.cdiv(m, _LANES)
    grid = pl.cdiv(tb, _NB)
    return pl.pallas_call(
        _tile_body,
        grid=(grid,),
        in_specs=[pl.BlockSpec((d, _LANES * _NB), lambda i: (0, i))],
        out_specs=pl.BlockSpec((_SUB, _NB, _SUB, _LANES),
                               lambda i: (0, i, 0, 0)),
        out_shape=jax.ShapeDtypeStruct((_SUB, tb, _SUB, _LANES), jnp.float32),
        compiler_params=pltpu.CompilerParams(
            dimension_semantics=("arbitrary",)),
    )(t_t)


def _untile_body(z_ref, o_ref):
    nb = z_ref.shape[1]
    for a in range(_SUB):
        za = z_ref[a, :, :, :]
        o_ref[_SUB * a:_SUB * (a + 1), :] = (
            jnp.swapaxes(za, 0, 1).reshape(_SUB, nb * _LANES))


def _from_tiles(z, m):
    s, tb, s2, lanes = z.shape
    d = s * s2
    grid = pl.cdiv(tb, _NB)
    return pl.pallas_call(
        _untile_body,
        grid=(grid,),
        in_specs=[pl.BlockSpec((_SUB, _NB, _SUB, _LANES),
                               lambda i: (0, i, 0, 0))],
        out_specs=pl.BlockSpec((d, _LANES * _NB), lambda i: (0, i)),
        out_shape=jax.ShapeDtypeStruct((d, m), jnp.float32),
        compiler_params=pltpu.CompilerParams(
            dimension_semantics=("arbitrary",)),
    )(z)


_JB = 512


def _delta_body(idx_col_ref, idx_row_ref, valt_ref, o_ref):
    eq = idx_col_ref[...] == idx_row_ref[...]
    e = jnp.where(eq, jnp.float32(2.0), jnp.float32(0.0))
    o_ref[...] = lax.dot_general(
        valt_ref[...], e, (((1,), (0,)), ((), ())),
        precision=lax.Precision.HIGHEST,
        preferred_element_type=jnp.float32,
    )


def _delta_t(indices, val_t):
    d, b = val_t.shape
    jb = _JB if b % _JB == 0 else b
    grid = b // jb
    idx_col = indices.reshape(b, 1)
    idx_row = indices.reshape(1, b)
    return pl.pallas_call(
        _delta_body,
        grid=(grid,),
        in_specs=[
            pl.BlockSpec((b, 1), lambda i: (0, 0)),
            pl.BlockSpec((1, jb), lambda i: (0, i)),
            pl.BlockSpec((d, b), lambda i: (0, 0)),
        ],
        out_specs=pl.BlockSpec((d, jb), lambda i: (0, i)),
        out_shape=jax.ShapeDtypeStruct((d, b), jnp.float32),
        compiler_params=pltpu.CompilerParams(
            dimension_semantics=("arbitrary",)),
    )(idx_col, idx_row, val_t)


_IDXW = 128


def _make_sc_gather(a_stride):
    def _body(zflat, deltat_hbm, idx_hbm, newrows_hbm, idxv, base, addr,
              words, dv, sem):
        c = lax.axis_index("c")
        s = lax.axis_index("s")
        w = s * 2 + c

        pltpu.sync_copy(idx_hbm.at[pl.ds(w, 1)], idxv)
        for j in range(_IDXW // 16):
            sl = pl.ds(16 * j, 16)
            v = idxv[0, sl]
            base[sl] = (lax.shift_right_logical(v, 7) * 1024) + (v & 127)
        pltpu.sync_copy(deltat_hbm.at[:, pl.ds(w * _IDXW, _IDXW)], dv)

        def fill(aq, _):
            off = (aq // 8) * a_stride + (aq % 8) * 128
            for j in range(_IDXW // 16):
                sl = pl.ds(16 * j, 16)
                addr[aq, sl] = base[sl] + off
            return 0

        lax.fori_loop(0, 64, fill, 0)

        @plsc.parallel_loop(0, 64, unroll=8)
        def gat(aq):
            pltpu.async_copy(zflat.at[addr.at[aq]], words.at[aq], sem)

        pltpu.make_async_copy(deltat_hbm.at[:, 0:_IDXW], words, sem).wait()

        def add(aq, _):
            for j in range(_IDXW // 16):
                sl = pl.ds(16 * j, 16)
                words[aq, sl] = words[aq, sl] + dv[aq, sl]
            return 0

        lax.fori_loop(0, 64, add, 0)
        pltpu.sync_copy(words, newrows_hbm.at[:, pl.ds(w * _IDXW, _IDXW)])

    return _body


def _make_sc_scatter(a_stride):
    def _body(zref, newrows_hbm, idx_hbm, idxv, base, addr, words, sem):
        c = lax.axis_index("c")
        s = lax.axis_index("s")
        w = s * 2 + c

        pltpu.sync_copy(idx_hbm.at[pl.ds(w, 1)], idxv)
        for j in range(_IDXW // 16):
            sl = pl.ds(16 * j, 16)
            v = idxv[0, sl]
            base[sl] = (lax.shift_right_logical(v, 7) * 1024) + (v & 127)
        pltpu.sync_copy(newrows_hbm.at[:, pl.ds(w * _IDXW, _IDXW)], words)

        def fill(aq, _):
            off = (aq // 8) * a_stride + (aq % 8) * 128
            for j in range(_IDXW // 16):
                sl = pl.ds(16 * j, 16)
                addr[aq, sl] = base[sl] + off
            return 0

        lax.fori_loop(0, 64, fill, 0)

        @plsc.parallel_loop(0, 64, unroll=8)
        def sca(aq):
            pltpu.async_copy(words.at[aq], zref.at[addr.at[aq]], sem)

        pltpu.make_async_copy(newrows_hbm.at[:, 0:_IDXW], words, sem).wait()

    return _body


def _sc_scatter(zref, zflat_in, delta_t, indices, tb):
    d, b = delta_t.shape
    idx2d = indices.reshape(b // _IDXW, _IDXW)
    mesh = plsc.VectorSubcoreMesh(
        core_axis_name="c", subcore_axis_name="s", num_cores=2, num_subcores=16)
    gather = pl.kernel(
        _make_sc_gather(tb * 1024),
        out_type=jax.ShapeDtypeStruct((d, b), jnp.float32),
        mesh=mesh,
        scratch_types=[
            pltpu.VMEM((1, _IDXW), jnp.int32),
            pltpu.VMEM((_IDXW,), jnp.int32),
            pltpu.VMEM((64, _IDXW), jnp.int32),
            pltpu.VMEM((64, _IDXW), jnp.float32),
            pltpu.VMEM((64, _IDXW), jnp.float32),
            pltpu.SemaphoreType.DMA,
        ],
        compiler_params=pltpu.CompilerParams(use_tc_tiling_on_sc=False),
    )
    newrows = gather(zflat_in, delta_t, idx2d)
    scatter = pl.kernel(
        _make_sc_scatter(tb * 1024),
        out_type=(),
        mesh=mesh,
        scratch_types=[
            pltpu.VMEM((1, _IDXW), jnp.int32),
            pltpu.VMEM((_IDXW,), jnp.int32),
            pltpu.VMEM((64, _IDXW), jnp.int32),
            pltpu.VMEM((64, _IDXW), jnp.float32),
            pltpu.SemaphoreType.DMA,
        ],
        compiler_params=pltpu.CompilerParams(use_tc_tiling_on_sc=False),
    )
    scatter(zref, newrows, idx2d)


def kernel(tensor, val, indices):
    m, d = tensor.shape
    tb = (m + _LANES - 1) // _LANES
    z = _to_tiles(tensor.T)
    delta_t = _delta_t(indices, val.T)
    zflat = z.reshape(-1)
    ref = jax.new_ref(zflat)
    _sc_scatter(ref, zflat, delta_t, indices, tb)
    zs = jax.freeze(ref).reshape(8, tb, 8, _LANES)
    out_t = _from_tiles(zs, m)
    return out_t.T
